# Initial kernel scaffold; baseline (speedup 1.0000x reference)
import functools

import jax
import jax.numpy as jnp
from jax import lax
from jax.experimental import pallas as pl
from jax.experimental.pallas import tpu as pltpu
from jax.experimental.pallas import tpu_sc as plsc

B = 16384
D = 64
NF = 8
EN = 12
NROWS = NF * EN
L = 16
NQ = D // L


def _prep_body(mu_ref, std_ref, act_ref, spt_ref, mud_ref):
    std = std_ref[...]
    spt_ref[...] = 0.01 * jnp.log(1.0 + jnp.exp(std))
    mud_ref[...] = lax.dot_general(
        act_ref[...], mu_ref[...], (((1,), (1,)), ((), ())),
        preferred_element_type=jnp.float32)


def _prep(mu_flat, std_flat, act):
    return pl.pallas_call(
        _prep_body,
        out_shape=(
            jax.ShapeDtypeStruct((NROWS, D), jnp.float32),
            jax.ShapeDtypeStruct((2, NROWS), jnp.float32),
        ),
    )(mu_flat, std_flat, act)


@functools.cache
def _build_sc():
    info = plsc.get_sparse_core_info()
    nc, ns = info.num_cores, info.num_subcores
    nw = nc * ns
    bw = B // nw
    nt = bw // L
    mesh = plsc.VectorSubcoreMesh(core_axis_name="c", subcore_axis_name="s")

    @functools.partial(
        pl.kernel,
        out_type=(
            jax.ShapeDtypeStruct((B,), jnp.float32),
            jax.ShapeDtypeStruct((B,), jnp.float32),
        ),
        mesh=mesh,
        scratch_types=[
            pltpu.VMEM((NF, bw), jnp.int32),
            pltpu.VMEM((bw, D), jnp.float32),
            pltpu.VMEM((NROWS, D), jnp.float32),
            pltpu.VMEM((NROWS,), jnp.float32),
            pltpu.VMEM((NROWS,), jnp.float32),
            pltpu.VMEM((2, D), jnp.float32),
            pltpu.VMEM((bw, L), jnp.float32),
            pltpu.VMEM((bw, L), jnp.float32),
            pltpu.VMEM((bw,), jnp.float32),
            pltpu.VMEM((bw,), jnp.float32),
        ],
    )
    def fm_sc(idx_hbm, v_hbm, spt_hbm, mud0_hbm, mud1_hbm, act_hbm,
              o0_hbm, o1_hbm,
              idx_v, v_v, spt_v, mud0_v, mud1_v, act_v, p0_v, p1_v,
              o0_v, o1_v):
        wid = lax.axis_index("s") * nc + lax.axis_index("c")
        base = wid * bw
        pltpu.sync_copy(idx_hbm.at[:, pl.ds(base, bw)], idx_v)
        pltpu.sync_copy(v_hbm.at[pl.ds(base, bw)], v_v)
        pltpu.sync_copy(spt_hbm, spt_v)
        pltpu.sync_copy(mud0_hbm, mud0_v)
        pltpu.sync_copy(mud1_hbm, mud1_v)
        pltpu.sync_copy(act_hbm, act_v)

        a0 = [act_v[0, pl.ds(q * L, L)] for q in range(NQ)]
        a1 = [act_v[1, pl.ds(q * L, L)] for q in range(NQ)]

        def row_body(b, carry):
            f = [idx_v[i, b] + i * EN for i in range(NF)]
            p0 = None
            p1 = None
            for q in range(NQ):
                g = spt_v[f[0], pl.ds(q * L, L)]
                for i in range(1, NF):
                    g = g + spt_v[f[i], pl.ds(q * L, L)]
                m = g * v_v[b, pl.ds(q * L, L)]
                t0 = m * a0[q]
                t1 = m * a1[q]
                p0 = t0 if q == 0 else p0 + t0
                p1 = t1 if q == 0 else p1 + t1
            p0_v[b] = p0
            p1_v[b] = p1
            return carry

        lax.fori_loop(0, bw, row_body, 0)

        iota = lax.iota(jnp.int32, L)

        def red_body(t, carry):
            row = t * L + iota
            acc0 = plsc.load_gather(p0_v, [row, jnp.zeros((L,), jnp.int32)])
            acc1 = plsc.load_gather(p1_v, [row, jnp.zeros((L,), jnp.int32)])
            for j in range(1, L):
                col = jnp.full((L,), j, jnp.int32)
                acc0 = acc0 + plsc.load_gather(p0_v, [row, col])
                acc1 = acc1 + plsc.load_gather(p1_v, [row, col])
            for i in range(NF):
                fi = idx_v[i, pl.ds(t * L, L)] + i * EN
                acc0 = acc0 + plsc.load_gather(mud0_v, [fi])
                acc1 = acc1 + plsc.load_gather(mud1_v, [fi])
            o0_v[pl.ds(t * L, L)] = acc0
            o1_v[pl.ds(t * L, L)] = acc1
            return carry

        lax.fori_loop(0, nt, red_body, 0)

        pltpu.sync_copy(o0_v, o0_hbm.at[pl.ds(base, bw)])
        pltpu.sync_copy(o1_v, o1_hbm.at[pl.ds(base, bw)])

    return fm_sc


def kernel(workclass, education, marital_status, occupation, relationship,
           race, sex, native_country, label, mean_tables, std_tables,
           action_table, rand_array):
    idx = jnp.stack([workclass, education, marital_status, occupation,
                     relationship, race, sex, native_country], axis=0)
    v = rand_array[: B * D].reshape(B, D)
    mu_flat = mean_tables.reshape(NROWS, D)
    std_flat = std_tables.reshape(NROWS, D)
    spt, mud = _prep(mu_flat, std_flat, action_table)
    o0, o1 = _build_sc()(idx, v, spt, mud[0], mud[1], action_table)
    return jnp.stack([o0, o1], axis=1)


# speedup vs baseline: 16.7442x; 16.7442x over previous
import functools

import jax
import jax.numpy as jnp
from jax import lax
from jax.experimental import pallas as pl
from jax.experimental.pallas import tpu as pltpu
from jax.experimental.pallas import tpu_sc as plsc

B = 16384
D = 64
NF = 8
EN = 12
NROWS = NF * EN
L = 16
NQ = D // L


def _prep_body(mu_ref, std_ref, act_ref, spt_ref, mud_ref):
    std = std_ref[...]
    spt_ref[...] = 0.01 * jnp.log(1.0 + jnp.exp(std))
    mud_ref[...] = lax.dot_general(
        act_ref[...], mu_ref[...], (((1,), (1,)), ((), ())),
        preferred_element_type=jnp.float32)


def _prep(mu_flat, std_flat, act):
    return pl.pallas_call(
        _prep_body,
        out_shape=(
            jax.ShapeDtypeStruct((NROWS, D), jnp.float32),
            jax.ShapeDtypeStruct((2, NROWS), jnp.float32),
        ),
    )(mu_flat, std_flat, act)


@functools.cache
def _build_sc():
    info = plsc.get_sparse_core_info()
    nc, ns = info.num_cores, info.num_subcores
    nw = nc * ns
    bw = B // nw
    nt = bw // L
    mesh = plsc.VectorSubcoreMesh(core_axis_name="c", subcore_axis_name="s")

    @functools.partial(
        pl.kernel,
        out_type=(
            jax.ShapeDtypeStruct((B,), jnp.float32),
            jax.ShapeDtypeStruct((B,), jnp.float32),
        ),
        mesh=mesh,
        compiler_params=pltpu.CompilerParams(needs_layout_passes=False),
        scratch_types=[
            pltpu.VMEM((NF, bw), jnp.int32),
            pltpu.VMEM((bw, D), jnp.float32),
            pltpu.VMEM((NROWS, D), jnp.float32),
            pltpu.VMEM((NROWS,), jnp.float32),
            pltpu.VMEM((NROWS,), jnp.float32),
            pltpu.VMEM((2, D), jnp.float32),
            pltpu.VMEM((bw * L,), jnp.float32),
            pltpu.VMEM((bw * L,), jnp.float32),
            pltpu.VMEM((bw,), jnp.float32),
            pltpu.VMEM((bw,), jnp.float32),
        ],
    )
    def fm_sc(idx_hbm, v_hbm, spt_hbm, mud0_hbm, mud1_hbm, act_hbm,
              o0_hbm, o1_hbm,
              idx_v, v_v, spt_v, mud0_v, mud1_v, act_v, p0_v, p1_v,
              o0_v, o1_v):
        wid = lax.axis_index("s") * nc + lax.axis_index("c")
        base = wid * bw
        pltpu.sync_copy(idx_hbm.at[:, pl.ds(base, bw)], idx_v)
        pltpu.sync_copy(v_hbm.at[pl.ds(base, bw)], v_v)
        pltpu.sync_copy(spt_hbm, spt_v)
        pltpu.sync_copy(mud0_hbm, mud0_v)
        pltpu.sync_copy(mud1_hbm, mud1_v)
        pltpu.sync_copy(act_hbm, act_v)

        a0 = [act_v[0, pl.ds(q * L, L)] for q in range(NQ)]
        a1 = [act_v[1, pl.ds(q * L, L)] for q in range(NQ)]

        def grp_body(t, carry):
            ivs = [idx_v[i, pl.ds(t * L, L)] for i in range(NF)]
            for k in range(L):
                b = t * L + k
                f = [ivs[i][k] + i * EN for i in range(NF)]
                p0 = None
                p1 = None
                for q in range(NQ):
                    g = spt_v[f[0], pl.ds(q * L, L)]
                    for i in range(1, NF):
                        g = g + spt_v[f[i], pl.ds(q * L, L)]
                    m = g * v_v[b, pl.ds(q * L, L)]
                    t0 = m * a0[q]
                    t1 = m * a1[q]
                    p0 = t0 if q == 0 else p0 + t0
                    p1 = t1 if q == 0 else p1 + t1
                p0_v[pl.ds(b * L, L)] = p0
                p1_v[pl.ds(b * L, L)] = p1
            return carry

        lax.fori_loop(0, nt, grp_body, 0)

        iota = lax.iota(jnp.int32, L)

        def red_body(t, carry):
            rowbase = (t * L + iota) * L
            acc0 = plsc.load_gather(p0_v, [rowbase])
            acc1 = plsc.load_gather(p1_v, [rowbase])
            for j in range(1, L):
                acc0 = acc0 + plsc.load_gather(p0_v, [rowbase + j])
                acc1 = acc1 + plsc.load_gather(p1_v, [rowbase + j])
            for i in range(NF):
                fi = idx_v[i, pl.ds(t * L, L)] + i * EN
                acc0 = acc0 + plsc.load_gather(mud0_v, [fi])
                acc1 = acc1 + plsc.load_gather(mud1_v, [fi])
            o0_v[pl.ds(t * L, L)] = acc0
            o1_v[pl.ds(t * L, L)] = acc1
            return carry

        lax.fori_loop(0, nt, red_body, 0)

        pltpu.sync_copy(o0_v, o0_hbm.at[pl.ds(base, bw)])
        pltpu.sync_copy(o1_v, o1_hbm.at[pl.ds(base, bw)])

    return fm_sc


def kernel(workclass, education, marital_status, occupation, relationship,
           race, sex, native_country, label, mean_tables, std_tables,
           action_table, rand_array):
    idx = jnp.stack([workclass, education, marital_status, occupation,
                     relationship, race, sex, native_country], axis=0)
    v = rand_array[: B * D].reshape(B, D)
    mu_flat = mean_tables.reshape(NROWS, D)
    std_flat = std_tables.reshape(NROWS, D)
    spt, mud = _prep(mu_flat, std_flat, action_table)
    o0, o1 = _build_sc()(idx, v, spt, mud[0], mud[1], action_table)
    return jnp.stack([o0, o1], axis=1)
